# baseline (speedup 1.0000x reference)
import functools

import jax
import jax.numpy as jnp
from jax import lax
from jax.experimental import pallas as pl
from jax.experimental.pallas import tpu as pltpu
from jax.experimental.pallas import tpu_sc as plsc

B = 128
V = 100000
N_FEAT = 25
TOPK = 10

L = 16
NC = 2
NS = 16
NW = NC * NS
ROWS_PER_W = B // NW
VPV = 8
GROUP = L * VPV
NG = (V + GROUP - 1) // GROUP
VPAD = NG * GROUP
SG = 16
NSG = (NG + SG - 1) // SG
NGP = NSG * SG
FPAD = 32
OPAD = 40
AV = 48
CAND = 2224
CAND_HI = 2048 - 144
NEG = float("-inf")


def _i32(x):
    return jnp.int32(x)


def _sort_asc(x):
    return plsc.sort_key_val(x, x)[0]


def _sort_desc(x):
    return plsc.sort_key_val(x, x, descending=True)[0]


def _topk_sc_body(item_hbm, feat_hbm, out_hbm, row_buf, gmax_buf, cand_buf,
                  av_buf, out_stage, sem, osem, q0, q1, q2, q3):
    qsems = (q0, q1, q2, q3)
    wid = lax.axis_index("s") * NC + lax.axis_index("c")
    ninf = jnp.full((L,), NEG, jnp.float32)
    iota = lax.iota(jnp.int32, L)

    for j in range(VPAD - V, 0, -L):
        row_buf[pl.ds(VPAD - j, L)] = ninf
    for g in range(NG, NGP):
        gmax_buf[pl.ds(g * L, L)] = ninf

    QB = (0, 196, 392, 588, NG)
    out_copies = []
    for r in range(ROWS_PER_W):
        row = wid * _i32(ROWS_PER_W) + _i32(r)
        rbase = row * _i32(VPAD)
        qcopies = []
        for q in range(4):
            lo_w = QB[q] * GROUP
            hi_w = min(QB[q + 1] * GROUP, V)
            qcopies.append(pltpu.async_copy(
                item_hbm.at[pl.ds(rbase + _i32(lo_w), hi_w - lo_w)],
                row_buf.at[pl.ds(lo_w, hi_w - lo_w)], qsems[q]))

        lane_max = ninf
        for q in range(4):
            qcopies[q].wait()

            @plsc.parallel_loop(_i32(QB[q]), _i32(QB[q + 1]), step=_i32(1),
                                unroll=4, carry=lane_max)
            def p1_loop(i, acc):
                base = i * _i32(GROUP)
                g0 = jnp.maximum(row_buf[pl.ds(base, L)],
                                 row_buf[pl.ds(base + L, L)])
                g1 = jnp.maximum(row_buf[pl.ds(base + 2 * L, L)],
                                 row_buf[pl.ds(base + 3 * L, L)])
                g2 = jnp.maximum(row_buf[pl.ds(base + 4 * L, L)],
                                 row_buf[pl.ds(base + 5 * L, L)])
                g3 = jnp.maximum(row_buf[pl.ds(base + 6 * L, L)],
                                 row_buf[pl.ds(base + 7 * L, L)])
                gm = jnp.maximum(jnp.maximum(g0, g1), jnp.maximum(g2, g3))
                gmax_buf[pl.ds(i * _i32(L), L)] = gm
                return jnp.maximum(acc, gm)

            lane_max = p1_loop

        lm_asc = _sort_asc(lane_max)
        t0 = lm_asc[6]
        tvec = jnp.full((L,), t0, jnp.float32)

        def _any_ge(v):
            cnt = plsc.all_reduce_population_count(v >= tvec)
            return cnt[0] > 0

        @plsc.parallel_loop(_i32(0), _i32(NSG), step=_i32(1), unroll=1,
                            carry=_i32(0))
        def p2_loop(i2, off):
            sbase = i2 * _i32(SG * L)
            m0 = jnp.maximum(gmax_buf[pl.ds(sbase, L)],
                             gmax_buf[pl.ds(sbase + L, L)])
            m1 = jnp.maximum(gmax_buf[pl.ds(sbase + 2 * L, L)],
                             gmax_buf[pl.ds(sbase + 3 * L, L)])
            m2 = jnp.maximum(gmax_buf[pl.ds(sbase + 4 * L, L)],
                             gmax_buf[pl.ds(sbase + 5 * L, L)])
            m3 = jnp.maximum(gmax_buf[pl.ds(sbase + 6 * L, L)],
                             gmax_buf[pl.ds(sbase + 7 * L, L)])
            m4 = jnp.maximum(gmax_buf[pl.ds(sbase + 8 * L, L)],
                             gmax_buf[pl.ds(sbase + 9 * L, L)])
            m5 = jnp.maximum(gmax_buf[pl.ds(sbase + 10 * L, L)],
                             gmax_buf[pl.ds(sbase + 11 * L, L)])
            m6 = jnp.maximum(gmax_buf[pl.ds(sbase + 12 * L, L)],
                             gmax_buf[pl.ds(sbase + 13 * L, L)])
            m7 = jnp.maximum(gmax_buf[pl.ds(sbase + 14 * L, L)],
                             gmax_buf[pl.ds(sbase + 15 * L, L)])
            mm = jnp.maximum(
                jnp.maximum(jnp.maximum(m0, m1), jnp.maximum(m2, m3)),
                jnp.maximum(jnp.maximum(m4, m5), jnp.maximum(m6, m7)))

            def descend(o0):
                def g_body(g, oo):
                    gm = gmax_buf[pl.ds(i2 * _i32(SG * L) + g * _i32(L), L)]

                    def filt(o):
                        def compact(oc):
                            plsc.store_scatter(cand_buf, [iota + oc], ninf,
                                               mask=iota == iota)
                            nv = jnp.right_shift(oc + _i32(L - 1), 4)

                            def m_body(h, tacc):
                                x = cand_buf[pl.ds(h * _i32(L), L)]
                                return _sort_asc(
                                    jnp.maximum(tacc, _sort_desc(x)))

                            tacc = lax.fori_loop(_i32(0), nv, m_body, ninf)
                            cand_buf[pl.ds(0, L)] = tacc
                            return _i32(L)

                        o = lax.cond(o > _i32(CAND_HI), compact,
                                     lambda oc: oc, o)
                        gbase = (i2 * _i32(SG) + g) * _i32(GROUP)
                        for j in range(VPV):
                            x = row_buf[pl.ds(gbase + j * L, L)]
                            msk = x >= tvec
                            plsc.store_compressed(cand_buf.at[pl.ds(o, L)],
                                                  x, mask=msk)
                            o = o + plsc.all_reduce_population_count(msk)[0]
                        return o

                    return lax.cond(_any_ge(gm), filt, lambda o: o, oo)

                return lax.fori_loop(_i32(0), _i32(SG), g_body, o0)

            return lax.cond(_any_ge(mm), descend, lambda o: o, off)

        noff = p2_loop
        plsc.store_scatter(cand_buf, [iota + noff], ninf, mask=iota == iota)
        nvec = jnp.right_shift(noff + _i32(L - 1), 4)

        def fin_body(h, tacc):
            x = cand_buf[pl.ds(h * _i32(L), L)]
            return _sort_asc(jnp.maximum(tacc, _sort_desc(x)))

        top_asc = lax.fori_loop(_i32(0), nvec, fin_body, ninf)

        pltpu.async_copy(feat_hbm.at[pl.ds(row * _i32(FPAD), FPAD)],
                         av_buf.at[pl.ds(0, FPAD)], sem).wait()
        av_buf[pl.ds(FPAD, L)] = ninf
        plsc.store_scatter(av_buf, [iota + _i32(N_FEAT)], jnp.flip(top_asc),
                           mask=iota < TOPK)
        a0 = av_buf[pl.ds(0, L)]
        a1 = av_buf[pl.ds(L, L)]
        a2 = av_buf[pl.ds(2 * L, L)]
        mx = jnp.max(jnp.maximum(jnp.maximum(a0, a1), a2))
        mv = jnp.full((L,), mx, jnp.float32)
        e0 = jnp.exp(a0 - mv)
        e1 = jnp.exp(a1 - mv)
        e2 = jnp.exp(a2 - mv)
        s = jnp.sum(e0 + e1 + e2)
        sv = jnp.full((L,), s, jnp.float32)
        ob = _i32(r * AV)
        out_stage[pl.ds(ob, L)] = e0 / sv
        out_stage[pl.ds(ob + L, L)] = e1 / sv
        out_stage[pl.ds(ob + 2 * L, L)] = e2 / sv
        out_copies.append(
            pltpu.async_copy(out_stage.at[pl.ds(ob, OPAD)],
                             out_hbm.at[pl.ds(row * _i32(OPAD), OPAD)], osem))
    for c in out_copies:
        c.wait()


_topk_sc = functools.partial(
    pl.kernel,
    out_type=jax.ShapeDtypeStruct((B * OPAD,), jnp.float32),
    mesh=plsc.VectorSubcoreMesh(core_axis_name="c", subcore_axis_name="s",
                                num_cores=NC, num_subcores=NS),
    compiler_params=pltpu.CompilerParams(needs_layout_passes=False,
                                         use_tc_tiling_on_sc=False),
    scratch_types=[
        pltpu.VMEM((VPAD,), jnp.float32),
        pltpu.VMEM((NGP * L,), jnp.float32),
        pltpu.VMEM((CAND,), jnp.float32),
        pltpu.VMEM((AV,), jnp.float32),
        pltpu.VMEM((ROWS_PER_W * AV,), jnp.float32),
        pltpu.SemaphoreType.DMA,
        pltpu.SemaphoreType.DMA,
        pltpu.SemaphoreType.DMA,
        pltpu.SemaphoreType.DMA,
        pltpu.SemaphoreType.DMA,
        pltpu.SemaphoreType.DMA,
    ],
)(_topk_sc_body)


def _relayout_body(in_ref, out_ref):
    for j in range(8):
        out_ref[pl.ds(j * VPAD, V)] = in_ref[j, :]


def kernel(item_scores, feat_scores, cand_item):
    item_flat = pl.pallas_call(
        _relayout_body,
        grid=(B // 8,),
        in_specs=[pl.BlockSpec((8, V), lambda i: (i, jnp.int32(0)))],
        out_specs=pl.BlockSpec((8 * VPAD,), lambda i: (i,)),
        out_shape=jax.ShapeDtypeStruct((B * VPAD,), jnp.float32),
    )(item_scores)
    feat_pad = jnp.pad(feat_scores, ((0, 0), (0, FPAD - N_FEAT))).reshape(-1)
    out = _topk_sc(item_flat, feat_pad)
    return out.reshape(B, OPAD)[:, :N_FEAT + TOPK]


# speedup vs baseline: 1.1372x; 1.1372x over previous
import functools

import jax
import jax.numpy as jnp
from jax import lax
from jax.experimental import pallas as pl
from jax.experimental.pallas import tpu as pltpu
from jax.experimental.pallas import tpu_sc as plsc

B = 128
V = 100000
N_FEAT = 25
TOPK = 10

L = 16
NC = 2
NS = 16
NW = NC * NS
ROWS_PER_W = B // NW
GROUP = 128
NG = 782
NGF = 781
VPAD = NG * GROUP + GROUP
GPAD = 1024
PADG = 782
FPAD = 32
OPAD = 40
AV = 48
BS = 16
CAND = 4384
CAND_HI = 2048
NEG = float("-inf")


def _i32(x):
    return jnp.int32(x)


def _sort_asc(x):
    return plsc.sort_key_val(x, x)[0]


def _sort_desc(x):
    return plsc.sort_key_val(x, x, descending=True)[0]


def _prep_body(in_ref, lin_ref, gmax_ref):
    x = in_ref[...]
    full = x[:, :NGF * GROUP].reshape(8, NGF, GROUP)
    gm = jnp.max(full, axis=2)
    tail = jnp.max(x[:, NGF * GROUP:], axis=1)
    for j in range(8):
        lin_ref[pl.ds(j * VPAD, V)] = x[j, :]
        lin_ref[pl.ds(j * VPAD + V, VPAD - V)] = jnp.full(
            (VPAD - V,), NEG, jnp.float32)
        grow = jnp.concatenate(
            [gm[j, :], tail[j][None],
             jnp.full((GPAD - NG,), NEG, jnp.float32)])
        gmax_ref[pl.ds(j * GPAD, GPAD)] = grow


def _topk_sc_body(item_hbm, gmax_hbm, feat_hbm, out_hbm,
                  gbuf, cidx, hbuf, cand_buf, av_buf, out_stage,
                  sem, osem, hsem):
    wid = lax.axis_index("s") * NC + lax.axis_index("c")
    ninf = jnp.full((L,), NEG, jnp.float32)
    iota = lax.iota(jnp.int32, L)
    pad_ids = jnp.full((L,), PADG, jnp.int32)

    out_copies = []
    for r in range(ROWS_PER_W):
        row = wid * _i32(ROWS_PER_W) + _i32(r)
        pltpu.async_copy(gmax_hbm.at[pl.ds(row * _i32(GPAD), GPAD)],
                         gbuf.at[pl.ds(0, GPAD)], sem).wait()

        @plsc.parallel_loop(_i32(0), _i32(GPAD // L), step=_i32(1),
                            unroll=4, carry=ninf)
        def lm_loop(i, acc):
            return jnp.maximum(acc, gbuf[pl.ds(i * _i32(L), L)])

        lm_asc = _sort_asc(lm_loop)
        t0 = lm_asc[6]
        tvec = jnp.full((L,), t0, jnp.float32)

        for k in range(0, GPAD + L, L):
            cidx[pl.ds(k, L)] = pad_ids

        @plsc.parallel_loop(_i32(0), _i32(GPAD // L), step=_i32(1),
                            unroll=1, carry=_i32(0))
        def hit_loop(i, hoff):
            v = gbuf[pl.ds(i * _i32(L), L)]
            msk = v >= tvec
            plsc.store_compressed(cidx.at[pl.ds(hoff, L)], iota + i * _i32(L),
                                  mask=msk)
            return hoff + plsc.all_reduce_population_count(msk)[0]

        hoff = hit_loop

        rbase = row * _i32(VPAD)
        nbatch = jnp.right_shift(hoff + _i32(BS - 1), 4)

        def b_body(b, off):
            idb = cidx[pl.ds(b * _i32(BS), L)]
            copies = []
            for k in range(BS):
                copies.append(pltpu.async_copy(
                    item_hbm.at[pl.ds(rbase + idb[k] * _i32(GROUP), GROUP)],
                    hbuf.at[pl.ds(k * GROUP, GROUP)], hsem))
            for c in copies:
                c.wait()

            def compact(oc):
                plsc.store_scatter(cand_buf, [iota + oc], ninf,
                                   mask=iota == iota)
                nv = jnp.right_shift(oc + _i32(L - 1), 4)

                def m_body(h, tacc):
                    xx = cand_buf[pl.ds(h * _i32(L), L)]
                    return _sort_asc(jnp.maximum(tacc, _sort_desc(xx)))

                tacc = lax.fori_loop(_i32(0), nv, m_body, ninf)
                cand_buf[pl.ds(0, L)] = tacc
                return _i32(L)

            off = lax.cond(off > _i32(CAND_HI), compact, lambda oc: oc, off)

            for k in range(BS * GROUP // L):
                xv = hbuf[pl.ds(k * L, L)]
                msk = xv >= tvec
                plsc.store_compressed(cand_buf.at[pl.ds(off, L)], xv,
                                      mask=msk)
                off = off + plsc.all_reduce_population_count(msk)[0]
            return off

        noff = lax.fori_loop(_i32(0), nbatch, b_body, _i32(0))

        plsc.store_scatter(cand_buf, [iota + noff], ninf, mask=iota == iota)
        nvec = jnp.right_shift(noff + _i32(L - 1), 4)

        def fin_body(h, tacc):
            x = cand_buf[pl.ds(h * _i32(L), L)]
            return _sort_asc(jnp.maximum(tacc, _sort_desc(x)))

        top_asc = lax.fori_loop(_i32(0), nvec, fin_body, ninf)

        pltpu.async_copy(feat_hbm.at[pl.ds(row * _i32(FPAD), FPAD)],
                         av_buf.at[pl.ds(0, FPAD)], sem).wait()
        av_buf[pl.ds(FPAD, L)] = ninf
        plsc.store_scatter(av_buf, [iota + _i32(N_FEAT)], jnp.flip(top_asc),
                           mask=iota < TOPK)
        a0 = av_buf[pl.ds(0, L)]
        a1 = av_buf[pl.ds(L, L)]
        a2 = av_buf[pl.ds(2 * L, L)]
        mx = jnp.max(jnp.maximum(jnp.maximum(a0, a1), a2))
        mv = jnp.full((L,), mx, jnp.float32)
        e0 = jnp.exp(a0 - mv)
        e1 = jnp.exp(a1 - mv)
        e2 = jnp.exp(a2 - mv)
        s = jnp.sum(e0 + e1 + e2)
        sv = jnp.full((L,), s, jnp.float32)
        ob = _i32(r * AV)
        out_stage[pl.ds(ob, L)] = e0 / sv
        out_stage[pl.ds(ob + L, L)] = e1 / sv
        out_stage[pl.ds(ob + 2 * L, L)] = e2 / sv
        out_copies.append(
            pltpu.async_copy(out_stage.at[pl.ds(ob, OPAD)],
                             out_hbm.at[pl.ds(row * _i32(OPAD), OPAD)], osem))
    for c in out_copies:
        c.wait()


_topk_sc = functools.partial(
    pl.kernel,
    out_type=jax.ShapeDtypeStruct((B * OPAD,), jnp.float32),
    mesh=plsc.VectorSubcoreMesh(core_axis_name="c", subcore_axis_name="s",
                                num_cores=NC, num_subcores=NS),
    compiler_params=pltpu.CompilerParams(needs_layout_passes=False,
                                         use_tc_tiling_on_sc=False),
    scratch_types=[
        pltpu.VMEM((GPAD,), jnp.float32),
        pltpu.VMEM((GPAD + 2 * L,), jnp.int32),
        pltpu.VMEM((BS * GROUP,), jnp.float32),
        pltpu.VMEM((CAND,), jnp.float32),
        pltpu.VMEM((AV,), jnp.float32),
        pltpu.VMEM((ROWS_PER_W * AV,), jnp.float32),
        pltpu.SemaphoreType.DMA,
        pltpu.SemaphoreType.DMA,
        pltpu.SemaphoreType.DMA,
    ],
)(_topk_sc_body)


def kernel(item_scores, feat_scores, cand_item):
    item_lin, gmax_flat = pl.pallas_call(
        _prep_body,
        grid=(B // 8,),
        in_specs=[pl.BlockSpec((8, V), lambda i: (i, jnp.int32(0)))],
        out_specs=[pl.BlockSpec((8 * VPAD,), lambda i: (i,)),
                   pl.BlockSpec((8 * GPAD,), lambda i: (i,))],
        out_shape=[jax.ShapeDtypeStruct((B * VPAD,), jnp.float32),
                   jax.ShapeDtypeStruct((B * GPAD,), jnp.float32)],
    )(item_scores)
    feat_pad = jnp.pad(feat_scores, ((0, 0), (0, FPAD - N_FEAT))).reshape(-1)
    out = _topk_sc(item_lin, gmax_flat, feat_pad)
    return out.reshape(B, OPAD)[:, :N_FEAT + TOPK]
